# baseline (speedup 1.0000x reference)
import functools

import jax
import jax.numpy as jnp
from jax import lax
from jax.experimental import pallas as pl
from jax.experimental.pallas import tpu as pltpu
from jax.experimental.pallas import tpu_sc as plsc

N_NODES = 10000
N_EDGES = 320000
IN_C = 128
OUT_C = 10
D = 16
NPAD = 10240
SLAB = NPAD // 16
CHUNK = 128
CT = N_EDGES // CHUNK
NTILES = 32
CPT_BASE = CT // NTILES
CPT_MAX = CPT_BASE + 1
K_OUT = 8
NBUF = 4

_mesh = plsc.VectorSubcoreMesh(core_axis_name="c", subcore_axis_name="s")
_sc_params = pltpu.CompilerParams(use_tc_tiling_on_sc=False)


def _tile_range(w):
    start = w * CPT_BASE + jnp.minimum(w, CT % NTILES)
    trips = CPT_BASE + (w < CT % NTILES).astype(jnp.int32)
    return start, trips


def _load_chunks(hbm, start, w, idx_v):
    pltpu.sync_copy(hbm.at[pl.ds(start, CPT_BASE)], idx_v.at[pl.ds(0, CPT_BASE)])

    @pl.when(w < CT % NTILES)
    def _():
        pltpu.sync_copy(hbm.at[pl.ds(start + CPT_BASE, 1)],
                        idx_v.at[pl.ds(CPT_BASE, 1)])


@functools.partial(
    pl.kernel,
    mesh=_mesh,
    out_type=jax.ShapeDtypeStruct((2 * NPAD,), jnp.float32),
    compiler_params=_sc_params,
    scratch_types=[
        pltpu.VMEM((CPT_MAX, CHUNK), jnp.int32),
        pltpu.VMEM((CHUNK,), jnp.float32),
        pltpu.VMEM((SLAB,), jnp.float32),
        pltpu.VMEM_SHARED((NPAD,), jnp.float32),
        pltpu.SemaphoreType.DMA,
    ],
)
def _sc_degree(dst_hbm, out_hbm, idx_v, ones_v, stage_v, hist_s, sem):
    c = lax.axis_index("c")
    s = lax.axis_index("s")
    w = c * 16 + s
    start, trips = _tile_range(w)

    for i in range(CHUNK // 16):
        ones_v[pl.ds(i * 16, 16)] = jnp.ones((16,), jnp.float32)
    for i in range(SLAB // 16):
        stage_v[pl.ds(i * 16, 16)] = jnp.zeros((16,), jnp.float32)

    pltpu.sync_copy(stage_v, hist_s.at[pl.ds(s * SLAB, SLAB)])
    plsc.subcore_barrier()

    _load_chunks(dst_hbm, start, w, idx_v)

    def body(j, carry):
        @pl.when(j >= K_OUT)
        def _():
            pltpu.make_async_copy(
                ones_v, hist_s.at[pl.ds(0, CHUNK)], sem).wait()
        pltpu.async_copy(ones_v, hist_s.at[idx_v.at[j]], sem, add=True)
        return carry

    lax.fori_loop(0, trips, body, 0)
    for _ in range(K_OUT):
        pltpu.make_async_copy(ones_v, hist_s.at[pl.ds(0, CHUNK)], sem).wait()
    plsc.subcore_barrier()

    pltpu.sync_copy(hist_s.at[pl.ds(s * SLAB, SLAB)], stage_v)
    pltpu.sync_copy(stage_v, out_hbm.at[pl.ds(c * NPAD + s * SLAB, SLAB)])


@functools.partial(
    pl.kernel,
    mesh=_mesh,
    out_type=jax.ShapeDtypeStruct((2 * NPAD, D), jnp.float32),
    compiler_params=_sc_params,
    scratch_types=[
        pltpu.VMEM((CPT_MAX, CHUNK), jnp.int32),
        pltpu.VMEM((CPT_MAX, CHUNK), jnp.int32),
        pltpu.VMEM((NBUF, CHUNK, D), jnp.float32),
        pltpu.VMEM((CHUNK, D), jnp.float32),
        pltpu.VMEM_SHARED((NPAD, D), jnp.float32),
        pltpu.SemaphoreType.DMA,
        pltpu.SemaphoreType.DMA,
        pltpu.SemaphoreType.DMA,
        pltpu.SemaphoreType.DMA,
        pltpu.SemaphoreType.DMA,
        pltpu.SemaphoreType.DMA,
        pltpu.SemaphoreType.DMA,
        pltpu.SemaphoreType.DMA,
    ],
)
def _sc_aggregate(y_hbm, src_hbm, dst_hbm, out_hbm,
                  isrc, idst, ring, stage, agg_s,
                  g0, g1, g2, g3, t0, t1, t2, t3):
    c = lax.axis_index("c")
    s = lax.axis_index("s")
    w = c * 16 + s
    start, trips = _tile_range(w)
    gsem = (g0, g1, g2, g3)
    tsem = (t0, t1, t2, t3)

    for i in range(CHUNK):
        stage[i] = jnp.zeros((16,), jnp.float32)
    for k in range(SLAB // CHUNK):
        pltpu.sync_copy(stage, agg_s.at[pl.ds(s * SLAB + k * CHUNK, CHUNK)])
    plsc.subcore_barrier()

    _load_chunks(src_hbm, start, w, isrc)
    _load_chunks(dst_hbm, start, w, idst)

    def wait_gather(b):
        pltpu.make_async_copy(y_hbm.at[isrc.at[0]], ring.at[b], gsem[b]).wait()

    def wait_scatter(b):
        pltpu.make_async_copy(
            ring.at[b], agg_s.at[pl.ds(0, CHUNK)], tsem[b]).wait()

    for b in range(NBUF):
        pltpu.async_copy(y_hbm.at[isrc.at[b]], ring.at[b], gsem[b])

    def group(g, carry):
        for b in range(NBUF):
            j = g * NBUF + b

            @pl.when(j < trips)
            def _(b=b, j=j):
                wait_gather(b)
                pltpu.async_copy(
                    ring.at[b], agg_s.at[idst.at[j]], tsem[b], add=True)
        for b in range(NBUF):
            j = g * NBUF + b

            @pl.when(j + NBUF < trips)
            def _(b=b, j=j):
                wait_scatter(b)
                pltpu.async_copy(
                    y_hbm.at[isrc.at[j + NBUF]], ring.at[b], gsem[b])
        return carry

    lax.fori_loop(0, (trips + NBUF - 1) // NBUF, group, 0)
    for b in range(NBUF):
        wait_scatter(b)
    plsc.subcore_barrier()

    for k in range(SLAB // CHUNK):
        pltpu.sync_copy(agg_s.at[pl.ds(s * SLAB + k * CHUNK, CHUNK)], stage)
        pltpu.sync_copy(
            stage, out_hbm.at[pl.ds(c * NPAD + s * SLAB + k * CHUNK, CHUNK)])


def _tc_matmul_body(x_ref, w_ref, xw_ref):
    xw_ref[...] = jnp.dot(x_ref[...], w_ref[...],
                          preferred_element_type=jnp.float32)


def _tc_matmul(x, w_pad):
    blk = 2000
    return pl.pallas_call(
        _tc_matmul_body,
        grid=(N_NODES // blk,),
        in_specs=[
            pl.BlockSpec((blk, IN_C), lambda i: (i, 0)),
            pl.BlockSpec((IN_C, D), lambda i: (0, 0)),
        ],
        out_specs=pl.BlockSpec((blk, D), lambda i: (i, 0)),
        out_shape=jax.ShapeDtypeStruct((N_NODES, D), jnp.float32),
    )(x, w_pad)


def _tc_scale_body(xw_ref, d0_ref, d1_ref, y_ref, dinv_ref):
    dinv = lax.rsqrt(d0_ref[...] + d1_ref[...] + 1.0)
    dinv_ref[...] = dinv
    y_ref[...] = xw_ref[...] * dinv


def _tc_scale(xw, d0, d1):
    return pl.pallas_call(
        _tc_scale_body,
        out_shape=[
            jax.ShapeDtypeStruct((N_NODES, D), jnp.float32),
            jax.ShapeDtypeStruct((N_NODES, 1), jnp.float32),
        ],
    )(xw, d0, d1)


def _tc_combine_body(agg_ref, y_ref, dinv_ref, b_ref, o_ref):
    a0 = agg_ref[pl.ds(0, N_NODES), :]
    a1 = agg_ref[pl.ds(NPAD, N_NODES), :]
    res = (a0 + a1 + y_ref[...]) * dinv_ref[...] + b_ref[...]
    o_ref[...] = res[:, :OUT_C]


def _tc_combine(agg_flat, y, dinv, b_pad):
    return pl.pallas_call(
        _tc_combine_body,
        out_shape=jax.ShapeDtypeStruct((N_NODES, OUT_C), jnp.float32),
    )(agg_flat, y, dinv, b_pad)


def kernel(x, edge_index, W, b):
    src2d = edge_index[0].astype(jnp.int32).reshape(CT, CHUNK)
    dst2d = edge_index[1].astype(jnp.int32).reshape(CT, CHUNK)

    w_pad = jnp.pad(W, ((0, 0), (0, D - W.shape[1])))
    deg_flat = _sc_degree(dst2d)
    xw = _tc_matmul(x, w_pad)

    d0 = deg_flat[:N_NODES].reshape(N_NODES, 1)
    d1 = deg_flat[NPAD:NPAD + N_NODES].reshape(N_NODES, 1)
    y, dinv = _tc_scale(xw, d0, d1)

    agg_flat = _sc_aggregate(y, src2d, dst2d)

    b_pad = jnp.pad(b, (0, D - b.shape[0])).reshape(1, D)
    return _tc_combine(agg_flat, y, dinv, b_pad)


# speedup vs baseline: 1.0102x; 1.0102x over previous
import functools

import jax
import jax.numpy as jnp
from jax import lax
from jax.experimental import pallas as pl
from jax.experimental.pallas import tpu as pltpu
from jax.experimental.pallas import tpu_sc as plsc

N_NODES = 10000
N_EDGES = 320000
IN_C = 128
OUT_C = 10
D = 16
NPAD = 10240
SLAB = NPAD // 16
CHUNK = 128
CT = N_EDGES // CHUNK
NTILES = 32
CPT_BASE = CT // NTILES
CPT_MAX = CPT_BASE + 1
K_OUT = 8
NBUF = 4

_mesh = plsc.VectorSubcoreMesh(core_axis_name="c", subcore_axis_name="s")
_sc_params = pltpu.CompilerParams(use_tc_tiling_on_sc=False)


def _tile_range(w):
    start = w * CPT_BASE + jnp.minimum(w, CT % NTILES)
    trips = CPT_BASE + (w < CT % NTILES).astype(jnp.int32)
    return start, trips


def _load_chunks(hbm, start, w, idx_v):
    pltpu.sync_copy(hbm.at[pl.ds(start, CPT_BASE)], idx_v.at[pl.ds(0, CPT_BASE)])

    @pl.when(w < CT % NTILES)
    def _():
        pltpu.sync_copy(hbm.at[pl.ds(start + CPT_BASE, 1)],
                        idx_v.at[pl.ds(CPT_BASE, 1)])


@functools.partial(
    pl.kernel,
    mesh=_mesh,
    out_type=jax.ShapeDtypeStruct((2 * NPAD,), jnp.float32),
    compiler_params=_sc_params,
    scratch_types=[
        pltpu.VMEM((CPT_MAX, CHUNK), jnp.int32),
        pltpu.VMEM((CHUNK,), jnp.float32),
        pltpu.VMEM((SLAB,), jnp.float32),
        pltpu.VMEM_SHARED((NPAD,), jnp.float32),
        pltpu.SemaphoreType.DMA,
    ],
)
def _sc_degree(dst_hbm, out_hbm, idx_v, ones_v, stage_v, hist_s, sem):
    c = lax.axis_index("c")
    s = lax.axis_index("s")
    w = c * 16 + s
    start, trips = _tile_range(w)

    for i in range(CHUNK // 16):
        ones_v[pl.ds(i * 16, 16)] = jnp.ones((16,), jnp.float32)
    for i in range(SLAB // 16):
        stage_v[pl.ds(i * 16, 16)] = jnp.zeros((16,), jnp.float32)

    pltpu.sync_copy(stage_v, hist_s.at[pl.ds(s * SLAB, SLAB)])
    plsc.subcore_barrier()

    _load_chunks(dst_hbm, start, w, idx_v)

    def body(j, carry):
        @pl.when(j >= K_OUT)
        def _():
            pltpu.make_async_copy(
                ones_v, hist_s.at[pl.ds(0, CHUNK)], sem).wait()
        pltpu.async_copy(ones_v, hist_s.at[idx_v.at[j]], sem, add=True)
        return carry

    lax.fori_loop(0, trips, body, 0)
    for _ in range(K_OUT):
        pltpu.make_async_copy(ones_v, hist_s.at[pl.ds(0, CHUNK)], sem).wait()
    plsc.subcore_barrier()

    pltpu.sync_copy(hist_s.at[pl.ds(s * SLAB, SLAB)], stage_v)
    pltpu.sync_copy(stage_v, out_hbm.at[pl.ds(c * NPAD + s * SLAB, SLAB)])


@functools.partial(
    pl.kernel,
    mesh=_mesh,
    out_type=jax.ShapeDtypeStruct((2 * NPAD, D), jnp.float32),
    compiler_params=_sc_params,
    scratch_types=[
        pltpu.VMEM((CPT_MAX, CHUNK), jnp.int32),
        pltpu.VMEM((CPT_MAX, CHUNK), jnp.int32),
        pltpu.VMEM((NBUF, CHUNK, D), jnp.float32),
        pltpu.VMEM((CHUNK, D), jnp.float32),
        pltpu.VMEM_SHARED((NPAD, D), jnp.float32),
        pltpu.SemaphoreType.DMA,
        pltpu.SemaphoreType.DMA,
        pltpu.SemaphoreType.DMA,
        pltpu.SemaphoreType.DMA,
        pltpu.SemaphoreType.DMA,
        pltpu.SemaphoreType.DMA,
        pltpu.SemaphoreType.DMA,
        pltpu.SemaphoreType.DMA,
    ],
)
def _sc_aggregate(y_hbm, src_hbm, dst_hbm, out_hbm,
                  isrc, idst, ring, stage, agg_s,
                  g0, g1, g2, g3, t0, t1, t2, t3):
    c = lax.axis_index("c")
    s = lax.axis_index("s")
    w = c * 16 + s
    start, trips = _tile_range(w)
    gsem = (g0, g1, g2, g3)
    tsem = (t0, t1, t2, t3)

    for i in range(CHUNK):
        stage[i] = jnp.zeros((16,), jnp.float32)
    for k in range(SLAB // CHUNK):
        pltpu.sync_copy(stage, agg_s.at[pl.ds(s * SLAB + k * CHUNK, CHUNK)])
    plsc.subcore_barrier()

    _load_chunks(src_hbm, start, w, isrc)
    _load_chunks(dst_hbm, start, w, idst)

    def wait_gather(b):
        pltpu.make_async_copy(y_hbm.at[isrc.at[0]], ring.at[b], gsem[b]).wait()

    def wait_scatter(b):
        pltpu.make_async_copy(
            ring.at[b], agg_s.at[pl.ds(0, CHUNK)], tsem[b]).wait()

    for b in range(NBUF):
        pltpu.async_copy(y_hbm.at[isrc.at[b]], ring.at[b], gsem[b])

    def group(g, carry):
        for b in range(NBUF):
            j = g * NBUF + b

            @pl.when(j < trips)
            def _(b=b, j=j):
                wait_gather(b)
                pltpu.async_copy(
                    ring.at[b], agg_s.at[idst.at[j]], tsem[b], add=True)
        for b in range(NBUF):
            j = g * NBUF + b

            @pl.when(j + NBUF < trips)
            def _(b=b, j=j):
                wait_scatter(b)
                pltpu.async_copy(
                    y_hbm.at[isrc.at[j + NBUF]], ring.at[b], gsem[b])
        return carry

    lax.fori_loop(0, (trips + NBUF - 1) // NBUF, group, 0)
    for b in range(NBUF):
        wait_scatter(b)
    plsc.subcore_barrier()

    for k in range(SLAB // CHUNK):
        pltpu.sync_copy(agg_s.at[pl.ds(s * SLAB + k * CHUNK, CHUNK)], stage)
        pltpu.sync_copy(
            stage, out_hbm.at[pl.ds(c * NPAD + s * SLAB + k * CHUNK, CHUNK)])


def _tc_linear_body(x_ref, w_ref, d0_ref, d1_ref, y_ref, dinv_ref):
    dinv = lax.rsqrt(d0_ref[...] + d1_ref[...] + 1.0)
    dinv_ref[...] = dinv
    y_ref[...] = dinv * jnp.dot(x_ref[...], w_ref[...],
                                preferred_element_type=jnp.float32)


def _tc_linear(x, w_pad, d0, d1):
    blk = 2000
    return pl.pallas_call(
        _tc_linear_body,
        grid=(N_NODES // blk,),
        in_specs=[
            pl.BlockSpec((blk, IN_C), lambda i: (i, 0)),
            pl.BlockSpec((IN_C, D), lambda i: (0, 0)),
            pl.BlockSpec((blk, 1), lambda i: (i, 0)),
            pl.BlockSpec((blk, 1), lambda i: (i, 0)),
        ],
        out_specs=[
            pl.BlockSpec((blk, D), lambda i: (i, 0)),
            pl.BlockSpec((blk, 1), lambda i: (i, 0)),
        ],
        out_shape=[
            jax.ShapeDtypeStruct((N_NODES, D), jnp.float32),
            jax.ShapeDtypeStruct((N_NODES, 1), jnp.float32),
        ],
    )(x, w_pad, d0, d1)


def _tc_combine_body(agg_ref, y_ref, dinv_ref, b_ref, o_ref):
    a0 = agg_ref[pl.ds(0, N_NODES), :]
    a1 = agg_ref[pl.ds(NPAD, N_NODES), :]
    res = (a0 + a1 + y_ref[...]) * dinv_ref[...] + b_ref[...]
    o_ref[...] = res[:, :OUT_C]


def _tc_combine(agg_flat, y, dinv, b_pad):
    return pl.pallas_call(
        _tc_combine_body,
        out_shape=jax.ShapeDtypeStruct((N_NODES, OUT_C), jnp.float32),
    )(agg_flat, y, dinv, b_pad)


def kernel(x, edge_index, W, b):
    src2d = edge_index[0].astype(jnp.int32).reshape(CT, CHUNK)
    dst2d = edge_index[1].astype(jnp.int32).reshape(CT, CHUNK)

    w_pad = jnp.pad(W, ((0, 0), (0, D - W.shape[1])))
    deg_flat = _sc_degree(dst2d)

    d0 = deg_flat[:N_NODES].reshape(N_NODES, 1)
    d1 = deg_flat[NPAD:NPAD + N_NODES].reshape(N_NODES, 1)
    y, dinv = _tc_linear(x, w_pad, d0, d1)

    agg_flat = _sc_aggregate(y, src2d, dst2d)

    b_pad = jnp.pad(b, (0, D - b.shape[0])).reshape(1, D)
    return _tc_combine(agg_flat, y, dinv, b_pad)


# speedup vs baseline: 1.1205x; 1.1092x over previous
import functools

import jax
import jax.numpy as jnp
from jax import lax
from jax.experimental import pallas as pl
from jax.experimental.pallas import tpu as pltpu
from jax.experimental.pallas import tpu_sc as plsc

N_NODES = 10000
N_EDGES = 320000
IN_C = 128
OUT_C = 10
D = 16
NPAD = 10240
SLAB = NPAD // 16
CHUNK = 128
CT = N_EDGES // CHUNK
NTILES = 32
CPT_BASE = CT // NTILES
CPT_MAX = CPT_BASE + 1
K_OUT = 8
NBUF = 4

_mesh = plsc.VectorSubcoreMesh(core_axis_name="c", subcore_axis_name="s")
_sc_params = pltpu.CompilerParams(use_tc_tiling_on_sc=False)


def _tile_range(w):
    start = w * CPT_BASE + jnp.minimum(w, CT % NTILES)
    trips = CPT_BASE + (w < CT % NTILES).astype(jnp.int32)
    return start, trips


def _load_chunks(edges_hbm, row, start, w, idx_v):
    pltpu.sync_copy(edges_hbm.at[row, pl.ds(start * CHUNK, CPT_BASE * CHUNK)],
                    idx_v.at[pl.ds(0, CPT_BASE * CHUNK)])

    @pl.when(w < CT % NTILES)
    def _():
        pltpu.sync_copy(
            edges_hbm.at[row, pl.ds((start + CPT_BASE) * CHUNK, CHUNK)],
            idx_v.at[pl.ds(CPT_BASE * CHUNK, CHUNK)])


@functools.partial(
    pl.kernel,
    mesh=_mesh,
    out_type=jax.ShapeDtypeStruct((2 * NPAD,), jnp.float32),
    compiler_params=_sc_params,
    scratch_types=[
        pltpu.VMEM((CPT_MAX * CHUNK,), jnp.int32),
        pltpu.VMEM((CHUNK,), jnp.float32),
        pltpu.VMEM((SLAB,), jnp.float32),
        pltpu.VMEM_SHARED((NPAD,), jnp.float32),
        pltpu.SemaphoreType.DMA,
    ],
)
def _sc_degree(edges_hbm, out_hbm, idx_v, ones_v, stage_v, hist_s, sem):
    c = lax.axis_index("c")
    s = lax.axis_index("s")
    w = c * 16 + s
    start, trips = _tile_range(w)

    for i in range(CHUNK // 16):
        ones_v[pl.ds(i * 16, 16)] = jnp.ones((16,), jnp.float32)
    for i in range(SLAB // 16):
        stage_v[pl.ds(i * 16, 16)] = jnp.zeros((16,), jnp.float32)

    pltpu.sync_copy(stage_v, hist_s.at[pl.ds(s * SLAB, SLAB)])
    plsc.subcore_barrier()

    _load_chunks(edges_hbm, 1, start, w, idx_v)

    def body(j, carry):
        @pl.when(j >= K_OUT)
        def _():
            pltpu.make_async_copy(
                ones_v, hist_s.at[pl.ds(0, CHUNK)], sem).wait()
        pltpu.async_copy(
            ones_v, hist_s.at[idx_v.at[pl.ds(j * CHUNK, CHUNK)]],
            sem, add=True)
        return carry

    lax.fori_loop(0, trips, body, 0)
    for _ in range(K_OUT):
        pltpu.make_async_copy(ones_v, hist_s.at[pl.ds(0, CHUNK)], sem).wait()
    plsc.subcore_barrier()

    pltpu.sync_copy(hist_s.at[pl.ds(s * SLAB, SLAB)], stage_v)
    pltpu.sync_copy(stage_v, out_hbm.at[pl.ds(c * NPAD + s * SLAB, SLAB)])


@functools.partial(
    pl.kernel,
    mesh=_mesh,
    out_type=jax.ShapeDtypeStruct((2 * NPAD, D), jnp.float32),
    compiler_params=_sc_params,
    scratch_types=[
        pltpu.VMEM((CPT_MAX * CHUNK,), jnp.int32),
        pltpu.VMEM((CPT_MAX * CHUNK,), jnp.int32),
        pltpu.VMEM((NBUF, CHUNK, D), jnp.float32),
        pltpu.VMEM((CHUNK, D), jnp.float32),
        pltpu.VMEM_SHARED((NPAD, D), jnp.float32),
        pltpu.SemaphoreType.DMA,
        pltpu.SemaphoreType.DMA,
        pltpu.SemaphoreType.DMA,
        pltpu.SemaphoreType.DMA,
        pltpu.SemaphoreType.DMA,
        pltpu.SemaphoreType.DMA,
        pltpu.SemaphoreType.DMA,
        pltpu.SemaphoreType.DMA,
    ],
)
def _sc_aggregate(y_hbm, edges_hbm, out_hbm,
                  isrc, idst, ring, stage, agg_s,
                  g0, g1, g2, g3, t0, t1, t2, t3):
    c = lax.axis_index("c")
    s = lax.axis_index("s")
    w = c * 16 + s
    start, trips = _tile_range(w)
    gsem = (g0, g1, g2, g3)
    tsem = (t0, t1, t2, t3)

    for i in range(CHUNK):
        stage[i] = jnp.zeros((16,), jnp.float32)
    for k in range(SLAB // CHUNK):
        pltpu.sync_copy(stage, agg_s.at[pl.ds(s * SLAB + k * CHUNK, CHUNK)])
    plsc.subcore_barrier()

    _load_chunks(edges_hbm, 0, start, w, isrc)
    _load_chunks(edges_hbm, 1, start, w, idst)

    def src_at(j):
        return isrc.at[pl.ds(j * CHUNK, CHUNK)]

    def dst_at(j):
        return idst.at[pl.ds(j * CHUNK, CHUNK)]

    def wait_gather(b):
        pltpu.make_async_copy(y_hbm.at[src_at(0)], ring.at[b], gsem[b]).wait()

    def wait_scatter(b):
        pltpu.make_async_copy(
            ring.at[b], agg_s.at[pl.ds(0, CHUNK)], tsem[b]).wait()

    for b in range(NBUF):
        pltpu.async_copy(y_hbm.at[src_at(b)], ring.at[b], gsem[b])

    def group(g, carry):
        for b in range(NBUF):
            j = g * NBUF + b

            @pl.when(j < trips)
            def _(b=b, j=j):
                wait_gather(b)
                pltpu.async_copy(
                    ring.at[b], agg_s.at[dst_at(j)], tsem[b], add=True)
        for b in range(NBUF):
            j = g * NBUF + b

            @pl.when(j + NBUF < trips)
            def _(b=b, j=j):
                wait_scatter(b)
                pltpu.async_copy(
                    y_hbm.at[src_at(j + NBUF)], ring.at[b], gsem[b])
        return carry

    lax.fori_loop(0, (trips + NBUF - 1) // NBUF, group, 0)
    for b in range(NBUF):
        wait_scatter(b)
    plsc.subcore_barrier()

    for k in range(SLAB // CHUNK):
        pltpu.sync_copy(agg_s.at[pl.ds(s * SLAB + k * CHUNK, CHUNK)], stage)
        pltpu.sync_copy(
            stage, out_hbm.at[pl.ds(c * NPAD + s * SLAB + k * CHUNK, CHUNK)])


def _tc_linear_body(x_ref, w_ref, d0_ref, d1_ref, y_ref, dinv_ref):
    dinv = lax.rsqrt(d0_ref[...] + d1_ref[...] + 1.0)
    dinv_ref[...] = dinv
    y_ref[...] = dinv * jnp.dot(x_ref[...], w_ref[...],
                                preferred_element_type=jnp.float32)


def _tc_linear(x, w_pad, d0, d1):
    blk = 2000
    return pl.pallas_call(
        _tc_linear_body,
        grid=(N_NODES // blk,),
        in_specs=[
            pl.BlockSpec((blk, IN_C), lambda i: (i, 0)),
            pl.BlockSpec((IN_C, D), lambda i: (0, 0)),
            pl.BlockSpec((blk, 1), lambda i: (i, 0)),
            pl.BlockSpec((blk, 1), lambda i: (i, 0)),
        ],
        out_specs=[
            pl.BlockSpec((blk, D), lambda i: (i, 0)),
            pl.BlockSpec((blk, 1), lambda i: (i, 0)),
        ],
        out_shape=[
            jax.ShapeDtypeStruct((N_NODES, D), jnp.float32),
            jax.ShapeDtypeStruct((N_NODES, 1), jnp.float32),
        ],
    )(x, w_pad, d0, d1)


def _tc_combine_body(agg_ref, y_ref, dinv_ref, b_ref, o_ref):
    a0 = agg_ref[pl.ds(0, N_NODES), :]
    a1 = agg_ref[pl.ds(NPAD, N_NODES), :]
    res = (a0 + a1 + y_ref[...]) * dinv_ref[...] + b_ref[...]
    o_ref[...] = res[:, :OUT_C]


def _tc_combine(agg_flat, y, dinv, b_pad):
    return pl.pallas_call(
        _tc_combine_body,
        out_shape=jax.ShapeDtypeStruct((N_NODES, OUT_C), jnp.float32),
    )(agg_flat, y, dinv, b_pad)


def kernel(x, edge_index, W, b):
    edges = edge_index.astype(jnp.int32)

    w_pad = jnp.pad(W, ((0, 0), (0, D - W.shape[1])))
    deg_flat = _sc_degree(edges)

    d0 = deg_flat[:N_NODES].reshape(N_NODES, 1)
    d1 = deg_flat[NPAD:NPAD + N_NODES].reshape(N_NODES, 1)
    y, dinv = _tc_linear(x, w_pad, d0, d1)

    agg_flat = _sc_aggregate(y, edges)

    b_pad = jnp.pad(b, (0, D - b.shape[0])).reshape(1, D)
    return _tc_combine(agg_flat, y, dinv, b_pad)


# speedup vs baseline: 1.5894x; 1.4184x over previous
import functools

import jax
import jax.numpy as jnp
from jax import lax
from jax.experimental import pallas as pl
from jax.experimental.pallas import tpu as pltpu
from jax.experimental.pallas import tpu_sc as plsc

N_NODES = 10000
N_EDGES = 320000
IN_C = 128
OUT_C = 10
D = 16
NPAD = 10240
SLAB = NPAD // 16
CHUNK = 128
CT = N_EDGES // CHUNK
NTILES = 32
CPT_BASE = CT // NTILES
CPT_MAX = CPT_BASE + 1
K_OUT = 8
NBUF = 4

_mesh = plsc.VectorSubcoreMesh(core_axis_name="c", subcore_axis_name="s")
_sc_params = pltpu.CompilerParams(use_tc_tiling_on_sc=False)


def _tile_range(w):
    start = w * CPT_BASE + jnp.minimum(w, CT % NTILES)
    trips = CPT_BASE + (w < CT % NTILES).astype(jnp.int32)
    return start, trips


def _load_chunks(edges_hbm, row, start, w, idx_v):
    pltpu.sync_copy(edges_hbm.at[row, pl.ds(start * CHUNK, CPT_BASE * CHUNK)],
                    idx_v.at[pl.ds(0, CPT_BASE * CHUNK)])

    @pl.when(w < CT % NTILES)
    def _():
        pltpu.sync_copy(
            edges_hbm.at[row, pl.ds((start + CPT_BASE) * CHUNK, CHUNK)],
            idx_v.at[pl.ds(CPT_BASE * CHUNK, CHUNK)])


@functools.partial(
    pl.kernel,
    mesh=_mesh,
    out_type=jax.ShapeDtypeStruct((2 * NPAD,), jnp.float32),
    compiler_params=_sc_params,
    scratch_types=[
        pltpu.VMEM((CPT_MAX * CHUNK,), jnp.int32),
        pltpu.VMEM((CHUNK,), jnp.float32),
        pltpu.VMEM((SLAB,), jnp.float32),
        pltpu.VMEM_SHARED((NPAD,), jnp.float32),
        pltpu.SemaphoreType.DMA,
    ],
)
def _sc_degree(edges_hbm, out_hbm, idx_v, ones_v, stage_v, hist_s, sem):
    c = lax.axis_index("c")
    s = lax.axis_index("s")
    w = c * 16 + s
    start, trips = _tile_range(w)

    for i in range(CHUNK // 16):
        ones_v[pl.ds(i * 16, 16)] = jnp.ones((16,), jnp.float32)
    for i in range(SLAB // 16):
        stage_v[pl.ds(i * 16, 16)] = jnp.zeros((16,), jnp.float32)

    pltpu.sync_copy(stage_v, hist_s.at[pl.ds(s * SLAB, SLAB)])
    plsc.subcore_barrier()

    _load_chunks(edges_hbm, 1, start, w, idx_v)

    def body(j, carry):
        @pl.when(j >= K_OUT)
        def _():
            pltpu.make_async_copy(
                ones_v, hist_s.at[pl.ds(0, CHUNK)], sem).wait()
        pltpu.async_copy(
            ones_v, hist_s.at[idx_v.at[pl.ds(j * CHUNK, CHUNK)]],
            sem, add=True)
        return carry

    lax.fori_loop(0, trips, body, 0)
    for _ in range(K_OUT):
        pltpu.make_async_copy(ones_v, hist_s.at[pl.ds(0, CHUNK)], sem).wait()
    plsc.subcore_barrier()

    pltpu.sync_copy(hist_s.at[pl.ds(s * SLAB, SLAB)], stage_v)
    pltpu.sync_copy(stage_v, out_hbm.at[pl.ds(c * NPAD + s * SLAB, SLAB)])


@functools.partial(
    pl.kernel,
    mesh=_mesh,
    out_type=jax.ShapeDtypeStruct((2 * NPAD, D), jnp.float32),
    compiler_params=_sc_params,
    scratch_types=[
        pltpu.VMEM((CPT_MAX * CHUNK,), jnp.int32),
        pltpu.VMEM((CPT_MAX * CHUNK,), jnp.int32),
        pltpu.VMEM((NBUF, CHUNK, D), jnp.float32),
        pltpu.VMEM((CHUNK, D), jnp.float32),
        pltpu.VMEM_SHARED((NPAD, D), jnp.float32),
        pltpu.SemaphoreType.DMA,
        pltpu.SemaphoreType.DMA,
        pltpu.SemaphoreType.DMA,
        pltpu.SemaphoreType.DMA,
        pltpu.SemaphoreType.DMA,
        pltpu.SemaphoreType.DMA,
        pltpu.SemaphoreType.DMA,
        pltpu.SemaphoreType.DMA,
    ],
)
def _sc_aggregate(y_hbm, edges_hbm, out_hbm,
                  isrc, idst, ring, stage, agg_s,
                  g0, g1, g2, g3, t0, t1, t2, t3):
    c = lax.axis_index("c")
    s = lax.axis_index("s")
    w = c * 16 + s
    start, trips = _tile_range(w)
    gsem = (g0, g1, g2, g3)
    tsem = (t0, t1, t2, t3)

    for i in range(CHUNK):
        stage[i] = jnp.zeros((16,), jnp.float32)
    for k in range(SLAB // CHUNK):
        pltpu.sync_copy(stage, agg_s.at[pl.ds(s * SLAB + k * CHUNK, CHUNK)])
    plsc.subcore_barrier()

    _load_chunks(edges_hbm, 0, start, w, isrc)
    _load_chunks(edges_hbm, 1, start, w, idst)

    def src_at(j):
        return isrc.at[pl.ds(j * CHUNK, CHUNK)]

    def dst_at(j):
        return idst.at[pl.ds(j * CHUNK, CHUNK)]

    def wait_gather(b):
        pltpu.make_async_copy(y_hbm.at[src_at(0)], ring.at[b], gsem[b]).wait()

    def wait_scatter(b):
        pltpu.make_async_copy(
            ring.at[b], agg_s.at[pl.ds(0, CHUNK)], tsem[b]).wait()

    for b in range(NBUF):
        pltpu.async_copy(y_hbm.at[src_at(b)], ring.at[b], gsem[b])

    def group(g, carry):
        for b in range(NBUF):
            j = g * NBUF + b

            @pl.when(j < trips)
            def _(b=b, j=j):
                wait_gather(b)
                pltpu.async_copy(
                    ring.at[b], agg_s.at[dst_at(j)], tsem[b], add=True)
        for b in range(NBUF):
            j = g * NBUF + b

            @pl.when(j + NBUF < trips)
            def _(b=b, j=j):
                wait_scatter(b)
                pltpu.async_copy(
                    y_hbm.at[src_at(j + NBUF)], ring.at[b], gsem[b])
        return carry

    lax.fori_loop(0, (trips + NBUF - 1) // NBUF, group, 0)
    for b in range(NBUF):
        wait_scatter(b)
    plsc.subcore_barrier()

    for k in range(SLAB // CHUNK):
        pltpu.sync_copy(agg_s.at[pl.ds(s * SLAB + k * CHUNK, CHUNK)], stage)
        pltpu.sync_copy(
            stage, out_hbm.at[pl.ds(c * NPAD + s * SLAB + k * CHUNK, CHUNK)])


PACK = N_NODES * D // 128


def _tc_linear_body(x8_ref, w_ref, drep_ref, y_ref):
    pieces = [
        jnp.dot(x8_ref[:, 128 * a:128 * (a + 1)], w_ref[...],
                preferred_element_type=jnp.float32)
        for a in range(8)
    ]
    xw128 = jnp.concatenate(pieces, axis=1)
    y_ref[...] = xw128 * lax.rsqrt(drep_ref[...] + 1.0)


def _tc_linear(x8, w_pad, drep):
    return pl.pallas_call(
        _tc_linear_body,
        out_shape=jax.ShapeDtypeStruct((PACK, 128), jnp.float32),
    )(x8, w_pad, drep)


def _tc_combine_body(agg_ref, y_ref, drep_ref, b_ref, o_ref):
    half = NPAD * D // 128
    a0 = agg_ref[:PACK, :]
    a1 = agg_ref[half:half + PACK, :]
    o_ref[...] = ((a0 + a1 + y_ref[...]) * lax.rsqrt(drep_ref[...] + 1.0)
                  + b_ref[...])


def _tc_combine(agg128, y128, drep, b_rep):
    return pl.pallas_call(
        _tc_combine_body,
        out_shape=jax.ShapeDtypeStruct((PACK, 128), jnp.float32),
    )(agg128, y128, drep, b_rep)


def kernel(x, edge_index, W, b):
    edges = edge_index.astype(jnp.int32)

    w_pad = jnp.pad(W, ((0, 0), (0, D - W.shape[1])))
    deg_flat = _sc_degree(edges)

    dsum = deg_flat[:N_NODES] + deg_flat[NPAD:NPAD + N_NODES]
    drep = jnp.repeat(dsum, D).reshape(PACK, 128)
    x8 = x.reshape(PACK, 8 * IN_C)
    y128 = _tc_linear(x8, w_pad, drep)

    agg_flat = _sc_aggregate(y128.reshape(N_NODES, D), edges)

    b_rep = jnp.tile(jnp.pad(b, (0, D - b.shape[0])), 128 // D).reshape(1, 128)
    o128 = _tc_combine(agg_flat.reshape(2 * NPAD * D // 128, 128), y128,
                       drep, b_rep)
    return o128.reshape(N_NODES, D)[:, :OUT_C]


# speedup vs baseline: 1.6889x; 1.0626x over previous
import functools

import jax
import jax.numpy as jnp
from jax import lax
from jax.experimental import pallas as pl
from jax.experimental.pallas import tpu as pltpu
from jax.experimental.pallas import tpu_sc as plsc

N_NODES = 10000
N_EDGES = 320000
IN_C = 128
OUT_C = 10
D = 16
NPAD = 10240
SLAB = NPAD // 16
CHUNK = 128
CT = N_EDGES // CHUNK
NTILES = 32
CPT_BASE = CT // NTILES
CPT_MAX = CPT_BASE + 1
K_OUT = 8
NBUF = 6

_mesh = plsc.VectorSubcoreMesh(core_axis_name="c", subcore_axis_name="s")
_sc_params = pltpu.CompilerParams(use_tc_tiling_on_sc=False)


def _tile_range(w):
    start = w * CPT_BASE + jnp.minimum(w, CT % NTILES)
    trips = CPT_BASE + (w < CT % NTILES).astype(jnp.int32)
    return start, trips


def _load_chunks(edges_hbm, row, start, w, idx_v):
    pltpu.sync_copy(edges_hbm.at[row, pl.ds(start * CHUNK, CPT_BASE * CHUNK)],
                    idx_v.at[pl.ds(0, CPT_BASE * CHUNK)])

    @pl.when(w < CT % NTILES)
    def _():
        pltpu.sync_copy(
            edges_hbm.at[row, pl.ds((start + CPT_BASE) * CHUNK, CHUNK)],
            idx_v.at[pl.ds(CPT_BASE * CHUNK, CHUNK)])


@functools.partial(
    pl.kernel,
    mesh=_mesh,
    out_type=jax.ShapeDtypeStruct((2 * NPAD,), jnp.float32),
    compiler_params=_sc_params,
    scratch_types=[
        pltpu.VMEM((CPT_MAX * CHUNK,), jnp.int32),
        pltpu.VMEM((CHUNK,), jnp.float32),
        pltpu.VMEM((SLAB,), jnp.float32),
        pltpu.VMEM_SHARED((NPAD,), jnp.float32),
        pltpu.SemaphoreType.DMA,
    ],
)
def _sc_degree(edges_hbm, out_hbm, idx_v, ones_v, stage_v, hist_s, sem):
    c = lax.axis_index("c")
    s = lax.axis_index("s")
    w = c * 16 + s
    start, trips = _tile_range(w)

    for i in range(CHUNK // 16):
        ones_v[pl.ds(i * 16, 16)] = jnp.ones((16,), jnp.float32)
    for i in range(SLAB // 16):
        stage_v[pl.ds(i * 16, 16)] = jnp.zeros((16,), jnp.float32)

    pltpu.sync_copy(stage_v, hist_s.at[pl.ds(s * SLAB, SLAB)])
    plsc.subcore_barrier()

    _load_chunks(edges_hbm, 1, start, w, idx_v)

    def body(j, carry):
        @pl.when(j >= K_OUT)
        def _():
            pltpu.make_async_copy(
                ones_v, hist_s.at[pl.ds(0, CHUNK)], sem).wait()
        pltpu.async_copy(
            ones_v, hist_s.at[idx_v.at[pl.ds(j * CHUNK, CHUNK)]],
            sem, add=True)
        return carry

    lax.fori_loop(0, trips, body, 0)
    for _ in range(K_OUT):
        pltpu.make_async_copy(ones_v, hist_s.at[pl.ds(0, CHUNK)], sem).wait()
    plsc.subcore_barrier()

    pltpu.sync_copy(hist_s.at[pl.ds(s * SLAB, SLAB)], stage_v)
    pltpu.sync_copy(stage_v, out_hbm.at[pl.ds(c * NPAD + s * SLAB, SLAB)])


@functools.partial(
    pl.kernel,
    mesh=_mesh,
    out_type=jax.ShapeDtypeStruct((2 * NPAD, D), jnp.float32),
    compiler_params=_sc_params,
    scratch_types=[
        pltpu.VMEM((CPT_MAX * CHUNK,), jnp.int32),
        pltpu.VMEM((CPT_MAX * CHUNK,), jnp.int32),
        pltpu.VMEM((NBUF, CHUNK, D), jnp.float32),
        pltpu.VMEM((CHUNK, D), jnp.float32),
        pltpu.VMEM_SHARED((NPAD, D), jnp.float32),
        pltpu.SemaphoreType.DMA,
        pltpu.SemaphoreType.DMA,
        pltpu.SemaphoreType.DMA,
        pltpu.SemaphoreType.DMA,
        pltpu.SemaphoreType.DMA,
        pltpu.SemaphoreType.DMA,
        pltpu.SemaphoreType.DMA,
        pltpu.SemaphoreType.DMA,
        pltpu.SemaphoreType.DMA,
        pltpu.SemaphoreType.DMA,
        pltpu.SemaphoreType.DMA,
        pltpu.SemaphoreType.DMA,
    ],
)
def _sc_aggregate(y_hbm, edges_hbm, out_hbm,
                  isrc, idst, ring, stage, agg_s,
                  g0, g1, g2, g3, g4, g5, t0, t1, t2, t3, t4, t5):
    c = lax.axis_index("c")
    s = lax.axis_index("s")
    w = c * 16 + s
    start, trips = _tile_range(w)
    gsem = (g0, g1, g2, g3, g4, g5)
    tsem = (t0, t1, t2, t3, t4, t5)

    for i in range(CHUNK):
        stage[i] = jnp.zeros((16,), jnp.float32)
    for k in range(SLAB // CHUNK):
        pltpu.sync_copy(stage, agg_s.at[pl.ds(s * SLAB + k * CHUNK, CHUNK)])
    plsc.subcore_barrier()

    _load_chunks(edges_hbm, 0, start, w, isrc)
    _load_chunks(edges_hbm, 1, start, w, idst)

    def src_at(j):
        return isrc.at[pl.ds(j * CHUNK, CHUNK)]

    def dst_at(j):
        return idst.at[pl.ds(j * CHUNK, CHUNK)]

    def wait_gather(b):
        pltpu.make_async_copy(y_hbm.at[src_at(0)], ring.at[b], gsem[b]).wait()

    def wait_scatter(b):
        pltpu.make_async_copy(
            ring.at[b], agg_s.at[pl.ds(0, CHUNK)], tsem[b]).wait()

    for b in range(NBUF):
        pltpu.async_copy(y_hbm.at[src_at(b)], ring.at[b], gsem[b])

    def group(g, carry):
        for b in range(NBUF):
            j = g * NBUF + b

            @pl.when(j < trips)
            def _(b=b, j=j):
                wait_gather(b)
                pltpu.async_copy(
                    ring.at[b], agg_s.at[dst_at(j)], tsem[b], add=True)
        for b in range(NBUF):
            j = g * NBUF + b

            @pl.when(j + NBUF < trips)
            def _(b=b, j=j):
                wait_scatter(b)
                pltpu.async_copy(
                    y_hbm.at[src_at(j + NBUF)], ring.at[b], gsem[b])
        return carry

    lax.fori_loop(0, (trips + NBUF - 1) // NBUF, group, 0)
    for b in range(NBUF):
        wait_scatter(b)
    plsc.subcore_barrier()

    for k in range(SLAB // CHUNK):
        pltpu.sync_copy(agg_s.at[pl.ds(s * SLAB + k * CHUNK, CHUNK)], stage)
        pltpu.sync_copy(
            stage, out_hbm.at[pl.ds(c * NPAD + s * SLAB + k * CHUNK, CHUNK)])


PACK = N_NODES * D // 128


def _tc_matmul_body(x8_ref, w_ref, xw_ref):
    pieces = [
        jnp.dot(x8_ref[:, 128 * a:128 * (a + 1)], w_ref[...],
                preferred_element_type=jnp.float32)
        for a in range(8)
    ]
    xw_ref[...] = jnp.concatenate(pieces, axis=1)


def _tc_matmul(x8, w_pad):
    return pl.pallas_call(
        _tc_matmul_body,
        out_shape=jax.ShapeDtypeStruct((PACK, 128), jnp.float32),
    )(x8, w_pad)


def _tc_scale_body(xw_ref, drep_ref, y_ref):
    y_ref[...] = xw_ref[...] * lax.rsqrt(drep_ref[...] + 1.0)


def _tc_scale(xw128, drep):
    return pl.pallas_call(
        _tc_scale_body,
        out_shape=jax.ShapeDtypeStruct((PACK, 128), jnp.float32),
    )(xw128, drep)


def _tc_combine_body(agg_ref, y_ref, drep_ref, b_ref, o_ref):
    half = NPAD * D // 128
    a0 = agg_ref[:PACK, :]
    a1 = agg_ref[half:half + PACK, :]
    o_ref[...] = ((a0 + a1 + y_ref[...]) * lax.rsqrt(drep_ref[...] + 1.0)
                  + b_ref[...])


def _tc_combine(agg128, y128, drep, b_rep):
    return pl.pallas_call(
        _tc_combine_body,
        out_shape=jax.ShapeDtypeStruct((PACK, 128), jnp.float32),
    )(agg128, y128, drep, b_rep)


def kernel(x, edge_index, W, b):
    edges = edge_index.astype(jnp.int32)

    w_pad = jnp.pad(W, ((0, 0), (0, D - W.shape[1])))
    deg_flat = _sc_degree(edges)

    dsum = deg_flat[:N_NODES] + deg_flat[NPAD:NPAD + N_NODES]
    drep = jnp.broadcast_to(dsum[:, None], (N_NODES, D)).reshape(PACK, 128)
    x8 = x.reshape(PACK, 8 * IN_C)
    xw128 = _tc_matmul(x8, w_pad)
    y128 = _tc_scale(xw128, drep)

    agg_flat = _sc_aggregate(y128.reshape(N_NODES, D), edges)

    b_rep = jnp.tile(jnp.pad(b, (0, D - b.shape[0])), 128 // D).reshape(1, 128)
    o128 = _tc_combine(agg_flat.reshape(2 * NPAD * D // 128, 128), y128,
                       drep, b_rep)
    return o128.reshape(N_NODES, D)[:, :OUT_C]
